# baseline (speedup 1.0000x reference)
import dataclasses
import functools
import math

import jax
import jax.numpy as jnp
from jax import lax
from jax.experimental import pallas as pl
from jax.experimental.pallas import tpu as pltpu
from jax.experimental.pallas import tpu_sc as plsc

NUM_CLASSES = 1000000
NUM_SAMPLED = 8192
BATCH = 4096
DIM = 64

_INV_LOG_RANGE = 1.0 / math.log(float(NUM_CLASSES + 1))

_CIN = 8064
_NSTEP = 31
_P = _CIN * _NSTEP
_TAIL = 4 * _P
_NHOLE = NUM_CLASSES - _TAIL

_NC = 2
_NS = 16
_L = 16
_NW = _NC * _NS
_NIDS = NUM_SAMPLED + BATCH
_PW = _NIDS // _NW
_CH = 128


def _sc_params(**overrides):
    cp = pltpu.CompilerParams()
    for f, v in overrides.items():
        if f in pltpu.CompilerParams.__dataclass_fields__:
            cp = dataclasses.replace(cp, **{f: v})
    return cp


def _relayout_body(p0_ref, p1_ref, p2_ref, p3_ref, h_ref, o_ref, h_out_ref):
    ident_bf = (lax.broadcasted_iota(jnp.int32, (DIM, DIM), 0)
                == lax.broadcasted_iota(jnp.int32, (DIM, DIM), 1)
                ).astype(jnp.bfloat16)

    def t(ref):
        return lax.dot_general(
            ref[...].astype(jnp.bfloat16), ident_bf, (((0,), (0,)), ((), ())),
            preferred_element_type=jnp.float32)

    t0, t1, t2, t3 = t(p0_ref), t(p1_ref), t(p2_ref), t(p3_ref)
    u = lambda x: lax.bitcast_convert_type(x, jnp.uint32)
    lo = lax.bitcast_convert_type((u(t0) >> 16) | u(t2), jnp.float32)
    hi = lax.bitcast_convert_type((u(t1) >> 16) | u(t3), jnp.float32)
    o_ref[:, :DIM] = lo
    o_ref[:, DIM:] = hi

    ident_f = (lax.broadcasted_iota(jnp.int32, (DIM, DIM), 0)
               == lax.broadcasted_iota(jnp.int32, (DIM, DIM), 1)
               ).astype(jnp.float32)
    th = lax.dot_general(h_ref[...][:, :DIM], ident_f, (((0,), (0,)), ((), ())),
                         preferred_element_type=jnp.float32)
    h_out_ref[:, :DIM] = th
    h_out_ref[:, DIM:] = th


def _tc_relayout(table_t):
    return pl.pallas_call(
        _relayout_body,
        grid=(_NSTEP,),
        compiler_params=_sc_params(dimension_semantics=("parallel",)),
        in_specs=[
            pl.BlockSpec((DIM, _CIN), lambda i: (0, i)),
            pl.BlockSpec((DIM, _CIN), lambda i: (0, i + _NSTEP)),
            pl.BlockSpec((DIM, _CIN), lambda i: (0, i + 2 * _NSTEP)),
            pl.BlockSpec((DIM, _CIN), lambda i: (0, i + 3 * _NSTEP)),
            pl.BlockSpec((DIM, 2 * DIM), lambda i: (0, _TAIL // (2 * DIM))),
        ],
        out_specs=[
            pl.BlockSpec((_CIN, 2 * DIM), lambda i: (i, 0)),
            pl.BlockSpec((_NHOLE, 2 * DIM), lambda i: (0, 0)),
        ],
        out_shape=[
            jax.ShapeDtypeStruct((_P, 2 * DIM), jnp.float32),
            jax.ShapeDtypeStruct((_NHOLE, 2 * DIM), jnp.float32),
        ],
    )(table_t, table_t, table_t, table_t, table_t)


def _sc_rows(packed, ids):
    mesh = plsc.VectorSubcoreMesh(core_axis_name="c", subcore_axis_name="s")

    @functools.partial(
        pl.kernel,
        mesh=mesh,
        out_type=jax.ShapeDtypeStruct((_NIDS, 2 * DIM), jnp.float32),
        scratch_types=[
            pltpu.VMEM((_PW,), jnp.int32),
            pltpu.VMEM((_PW,), jnp.int32),
            pltpu.VMEM((_PW, 2 * DIM), jnp.float32),
            pltpu.SemaphoreType.DMA,
        ],
    )
    def k(packed_hbm, ids_hbm, rows_out, idx_v, idmod_v, rows_v, sem):
        wid = lax.axis_index("s") * _NC + lax.axis_index("c")
        base = wid * _PW
        pltpu.sync_copy(ids_hbm.at[pl.ds(base, _PW)], idx_v)

        @pl.loop(0, _PW, step=_L)
        def _(c):
            v = idx_v[pl.ds(c, _L)]
            v = jnp.where(v >= 2 * _P, v - 2 * _P, v)
            v = jnp.where(v >= _P, v - _P, v)
            idmod_v[pl.ds(c, _L)] = jnp.where(v >= _P, 0, v)

        handles = []
        for j in range(_PW // _CH):
            sl = pl.ds(j * _CH, _CH)
            handles.append(
                pltpu.async_copy(packed_hbm.at[idmod_v.at[sl]], rows_v.at[sl], sem))
        for h in handles:
            h.wait()

        pltpu.sync_copy(rows_v, rows_out.at[pl.ds(base, _PW)])

    return k(packed, ids)


def _sc_bias(bias2d, ids):
    mesh = plsc.VectorSubcoreMesh(core_axis_name="c", subcore_axis_name="s")

    @functools.partial(
        pl.kernel,
        mesh=mesh,
        compiler_params=_sc_params(needs_layout_passes=False,
                                   use_tc_tiling_on_sc=False),
        out_type=jax.ShapeDtypeStruct((_NIDS,), jnp.float32),
        scratch_types=[
            pltpu.VMEM((_PW,), jnp.int32),
            pltpu.VMEM((_PW,), jnp.int32),
            pltpu.VMEM((_PW, _L), jnp.float32),
            pltpu.VMEM((_PW,), jnp.float32),
            pltpu.SemaphoreType.DMA,
        ],
    )
    def k(bias_hbm, ids_hbm, bias_out, idx_v, idxhi_v, brows_v, bvals_v, sem):
        wid = lax.axis_index("s") * _NC + lax.axis_index("c")
        base = wid * _PW
        pltpu.sync_copy(ids_hbm.at[pl.ds(base, _PW)], idx_v)

        @pl.loop(0, _PW, step=_L)
        def _(c):
            v = idx_v[pl.ds(c, _L)]
            idxhi_v[pl.ds(c, _L)] = lax.shift_right_logical(v, 4)

        handles = []
        for j in range(_PW // _CH):
            sl = pl.ds(j * _CH, _CH)
            handles.append(
                pltpu.async_copy(bias_hbm.at[idxhi_v.at[sl]], brows_v.at[sl], sem))
        for h in handles:
            h.wait()

        lane = jnp.arange(_L, dtype=jnp.int32)

        @pl.loop(0, _PW, step=_L)
        def _(c):
            lo = jnp.bitwise_and(idx_v[pl.ds(c, _L)], _L - 1)
            bvals_v[pl.ds(c, _L)] = plsc.load_gather(brows_v, [c + lane, lo])

        pltpu.sync_copy(bvals_v, bias_out.at[pl.ds(base, _PW)])

    return k(bias2d, ids)


_BLK = 256


def _log_expected_count(idsf):
    q = (jnp.log(idsf + 2.0) - jnp.log(idsf + 1.0)) * _INV_LOG_RANGE
    return jnp.log(q * float(NUM_SAMPLED))


def _select_rows(rows2, idc, hole_w):
    n = rows2.shape[0]
    q2 = idc >= 2 * _P
    rem = jnp.where(q2, idc - 2 * _P, idc)
    q1 = rem >= _P
    hl = jnp.where(q1, rows2[:, DIM:], rows2[:, :DIM])
    b = lax.bitcast_convert_type(hl, jnp.uint32)
    wb = jnp.where(q2, b & jnp.uint32(0xFFFF0000), b << 16)
    w = lax.bitcast_convert_type(wb, jnp.float32)
    oh = (lax.broadcasted_iota(jnp.int32, (n, _NHOLE), 1)
          == (idc - _TAIL)).astype(jnp.float32)
    hw = lax.dot_general(oh, hole_w, (((1,), (0,)), ((), ())),
                         preferred_element_type=jnp.float32)
    return jnp.where(idc >= _TAIL, hw, w)


def _unpack_body(sw2_ref, sidc_ref, hole_ref, sb_ref, out_ref):
    w = _select_rows(sw2_ref[...], sidc_ref[...], hole_ref[...][:, :DIM])
    idf = sidc_ref[...].astype(jnp.float32)
    bc = sb_ref[...] - _log_expected_count(idf)
    b0 = bc.astype(jnp.bfloat16).astype(jnp.float32)
    r1 = bc - b0
    b1 = r1.astype(jnp.bfloat16).astype(jnp.float32)
    b2 = r1 - b1
    col = lax.broadcasted_iota(jnp.int32, (NUM_SAMPLED, 2 * DIM), 1)
    aug = jnp.where(col < DIM, jnp.pad(w, ((0, 0), (0, DIM))),
                    jnp.where(col == DIM, b0,
                              jnp.where(col == DIM + 1, b1,
                                        jnp.where(col == DIM + 2, b2, 0.0))))
    out_ref[...] = aug.astype(jnp.bfloat16)


def _tc_unpack(rows2, sidc, hole_w, sbc):
    return pl.pallas_call(
        _unpack_body,
        grid=(1,),
        in_specs=[
            pl.BlockSpec((NUM_SAMPLED, 2 * DIM), lambda i: (0, 0)),
            pl.BlockSpec((NUM_SAMPLED, 1), lambda i: (0, 0)),
            pl.BlockSpec((_NHOLE, 2 * DIM), lambda i: (0, 0)),
            pl.BlockSpec((NUM_SAMPLED, 1), lambda i: (0, 0)),
        ],
        out_specs=pl.BlockSpec((NUM_SAMPLED, 2 * DIM), lambda i: (0, 0)),
        out_shape=jax.ShapeDtypeStruct((NUM_SAMPLED, 2 * DIM), jnp.bfloat16),
    )(rows2, sidc, hole_w, sbc)


def _tc_body(x_ref, tw2_ref, tb_ref, tid_ref, sw_ref, sid_ref,
             hole_ref, out_ref):
    sid = sid_ref[...]
    x = x_ref[...]
    logits = lax.dot_general(
        x, sw_ref[...], (((1,), (1,)), ((), ())),
        preferred_element_type=jnp.float32)
    tid = tid_ref[...]
    logits = jnp.where(tid == sid, logits - 1e9, logits)

    tw = _select_rows(tw2_ref[...], tid, hole_ref[...][:, :DIM])
    tl = (jnp.sum(x[:, :DIM].astype(jnp.float32) * tw, axis=1, keepdims=True)
          + tb_ref[...])
    tl = tl - _log_expected_count(tid.astype(jnp.float32))
    m = jnp.maximum(jnp.max(logits, axis=1, keepdims=True), tl)
    se = jnp.sum(jnp.exp(logits - m), axis=1, keepdims=True) + jnp.exp(tl - m)
    out_ref[...] = jnp.log(se) + m - tl


def _tc_loss(x_aug, rows2, tb, tid, sw_aug, sid, hole_w):
    ts = NUM_SAMPLED // _BLK
    return pl.pallas_call(
        _tc_body,
        grid=(BATCH // _BLK,),
        compiler_params=_sc_params(dimension_semantics=("parallel",)),
        in_specs=[
            pl.BlockSpec((_BLK, 2 * DIM), lambda i: (i, 0)),
            pl.BlockSpec((_BLK, 2 * DIM), lambda i: (i + ts, 0)),
            pl.BlockSpec((_BLK, 1), lambda i: (i, 0)),
            pl.BlockSpec((_BLK, 1), lambda i: (i, 0)),
            pl.BlockSpec((NUM_SAMPLED, 2 * DIM), lambda i: (0, 0)),
            pl.BlockSpec((1, NUM_SAMPLED), lambda i: (0, 0)),
            pl.BlockSpec((_NHOLE, 2 * DIM), lambda i: (0, 0)),
        ],
        out_specs=pl.BlockSpec((_BLK, 1), lambda i: (i, 0)),
        out_shape=jax.ShapeDtypeStruct((BATCH, 1), jnp.float32),
    )(x_aug, rows2, tb, tid, sw_aug, sid, hole_w)


def kernel(inputs, labels, kernel, bias, sampled_ids):
    table_t = kernel.T
    packed, hole_w = _tc_relayout(table_t)
    ids_all = jnp.concatenate([sampled_ids, labels[:, 0]])
    rows2 = _sc_rows(packed, ids_all)
    bvals = _sc_bias(bias.reshape(NUM_CLASSES // _L, _L), ids_all)
    sb = bvals[:NUM_SAMPLED].reshape(1, NUM_SAMPLED)
    tb = bvals[NUM_SAMPLED:].reshape(BATCH, 1)
    sid = sampled_ids.reshape(1, NUM_SAMPLED)
    sidc = sampled_ids.reshape(NUM_SAMPLED, 1)
    sw_aug = _tc_unpack(rows2, sidc, hole_w, sb.reshape(NUM_SAMPLED, 1))
    x_bf = inputs.astype(jnp.bfloat16)
    x_aug = jnp.concatenate(
        [x_bf, jnp.ones((BATCH, 3), jnp.bfloat16),
         jnp.zeros((BATCH, DIM - 3), jnp.bfloat16)], axis=1)
    out = _tc_loss(x_aug, rows2, tb, labels, sw_aug, sid, hole_w)
    return out[:, 0]


# speedup vs baseline: 1.0146x; 1.0146x over previous
import dataclasses
import functools
import math

import jax
import jax.numpy as jnp
from jax import lax
from jax.experimental import pallas as pl
from jax.experimental.pallas import tpu as pltpu
from jax.experimental.pallas import tpu_sc as plsc

NUM_CLASSES = 1000000
NUM_SAMPLED = 8192
BATCH = 4096
DIM = 64

_INV_LOG_RANGE = 1.0 / math.log(float(NUM_CLASSES + 1))

_CIN = 8064
_NSTEP = 31
_P = _CIN * _NSTEP
_TAIL = 4 * _P
_NHOLE = NUM_CLASSES - _TAIL

_NC = 2
_NS = 16
_L = 16
_NW = _NC * _NS
_NIDS = NUM_SAMPLED + BATCH
_PW = _NIDS // _NW
_CH = 128


def _sc_params(**overrides):
    cp = pltpu.CompilerParams()
    for f, v in overrides.items():
        if f in pltpu.CompilerParams.__dataclass_fields__:
            cp = dataclasses.replace(cp, **{f: v})
    return cp


def _relayout_body(p0_ref, p1_ref, p2_ref, p3_ref, h_ref, o_ref, h_out_ref):
    ident_bf = (lax.broadcasted_iota(jnp.int32, (DIM, DIM), 0)
                == lax.broadcasted_iota(jnp.int32, (DIM, DIM), 1)
                ).astype(jnp.bfloat16)

    def t(ref):
        return lax.dot_general(
            ref[...].astype(jnp.bfloat16), ident_bf, (((0,), (0,)), ((), ())),
            preferred_element_type=jnp.float32)

    t0, t1, t2, t3 = t(p0_ref), t(p1_ref), t(p2_ref), t(p3_ref)
    u = lambda x: lax.bitcast_convert_type(x, jnp.uint32)
    lo = lax.bitcast_convert_type((u(t0) >> 16) | u(t2), jnp.float32)
    hi = lax.bitcast_convert_type((u(t1) >> 16) | u(t3), jnp.float32)
    o_ref[:, :DIM] = lo
    o_ref[:, DIM:] = hi

    ident_f = (lax.broadcasted_iota(jnp.int32, (DIM, DIM), 0)
               == lax.broadcasted_iota(jnp.int32, (DIM, DIM), 1)
               ).astype(jnp.float32)
    th = lax.dot_general(h_ref[...][:, :DIM], ident_f, (((0,), (0,)), ((), ())),
                         preferred_element_type=jnp.float32)
    h_out_ref[:, :DIM] = th
    h_out_ref[:, DIM:] = th


def _tc_relayout(table_t):
    return pl.pallas_call(
        _relayout_body,
        grid=(_NSTEP,),
        compiler_params=_sc_params(dimension_semantics=("parallel",)),
        in_specs=[
            pl.BlockSpec((DIM, _CIN), lambda i: (0, i)),
            pl.BlockSpec((DIM, _CIN), lambda i: (0, i + _NSTEP)),
            pl.BlockSpec((DIM, _CIN), lambda i: (0, i + 2 * _NSTEP)),
            pl.BlockSpec((DIM, _CIN), lambda i: (0, i + 3 * _NSTEP)),
            pl.BlockSpec((DIM, 2 * DIM), lambda i: (0, _TAIL // (2 * DIM))),
        ],
        out_specs=[
            pl.BlockSpec((_CIN, 2 * DIM), lambda i: (i, 0)),
            pl.BlockSpec((_NHOLE, 2 * DIM), lambda i: (0, 0)),
        ],
        out_shape=[
            jax.ShapeDtypeStruct((_P, 2 * DIM), jnp.float32),
            jax.ShapeDtypeStruct((_NHOLE, 2 * DIM), jnp.float32),
        ],
    )(table_t, table_t, table_t, table_t, table_t)


def _sc_rows(packed, ids):
    mesh = plsc.VectorSubcoreMesh(core_axis_name="c", subcore_axis_name="s")

    @functools.partial(
        pl.kernel,
        mesh=mesh,
        out_type=jax.ShapeDtypeStruct((_NIDS, 2 * DIM), jnp.float32),
        scratch_types=[
            pltpu.VMEM((_PW,), jnp.int32),
            pltpu.VMEM((_PW,), jnp.int32),
            pltpu.VMEM((_PW, 2 * DIM), jnp.float32),
            pltpu.SemaphoreType.DMA,
        ],
    )
    def k(packed_hbm, ids_hbm, rows_out, idx_v, idmod_v, rows_v, sem):
        wid = lax.axis_index("s") * _NC + lax.axis_index("c")
        base = wid * _PW
        pltpu.sync_copy(ids_hbm.at[pl.ds(base, _PW)], idx_v)

        @pl.loop(0, _PW, step=_L)
        def _(c):
            v = idx_v[pl.ds(c, _L)]
            v = jnp.where(v >= 2 * _P, v - 2 * _P, v)
            v = jnp.where(v >= _P, v - _P, v)
            idmod_v[pl.ds(c, _L)] = jnp.where(v >= _P, 0, v)

        handles = []
        for j in range(_PW // _CH):
            sl = pl.ds(j * _CH, _CH)
            handles.append(
                pltpu.async_copy(packed_hbm.at[idmod_v.at[sl]], rows_v.at[sl], sem))
        for h in handles:
            h.wait()

        pltpu.sync_copy(rows_v, rows_out.at[pl.ds(base, _PW)])

    return k(packed, ids)


def _sc_bias(bias2d, ids):
    mesh = plsc.VectorSubcoreMesh(core_axis_name="c", subcore_axis_name="s")

    @functools.partial(
        pl.kernel,
        mesh=mesh,
        compiler_params=_sc_params(needs_layout_passes=False,
                                   use_tc_tiling_on_sc=False),
        out_type=jax.ShapeDtypeStruct((_NIDS,), jnp.float32),
        scratch_types=[
            pltpu.VMEM((_PW,), jnp.int32),
            pltpu.VMEM((_PW,), jnp.int32),
            pltpu.VMEM((_PW, _L), jnp.float32),
            pltpu.VMEM((_PW,), jnp.float32),
            pltpu.SemaphoreType.DMA,
        ],
    )
    def k(bias_hbm, ids_hbm, bias_out, idx_v, idxhi_v, brows_v, bvals_v, sem):
        wid = lax.axis_index("s") * _NC + lax.axis_index("c")
        base = wid * _PW
        pltpu.sync_copy(ids_hbm.at[pl.ds(base, _PW)], idx_v)

        @pl.loop(0, _PW, step=_L)
        def _(c):
            v = idx_v[pl.ds(c, _L)]
            idxhi_v[pl.ds(c, _L)] = lax.shift_right_logical(v, 4)

        handles = []
        for j in range(_PW // _CH):
            sl = pl.ds(j * _CH, _CH)
            handles.append(
                pltpu.async_copy(bias_hbm.at[idxhi_v.at[sl]], brows_v.at[sl], sem))
        for h in handles:
            h.wait()

        lane = jnp.arange(_L, dtype=jnp.int32)

        @pl.loop(0, _PW, step=_L)
        def _(c):
            lo = jnp.bitwise_and(idx_v[pl.ds(c, _L)], _L - 1)
            bvals_v[pl.ds(c, _L)] = plsc.load_gather(brows_v, [c + lane, lo])

        pltpu.sync_copy(bvals_v, bias_out.at[pl.ds(base, _PW)])

    return k(bias2d, ids)


_BLK = 256


def _log_expected_count(idsf):
    q = (jnp.log(idsf + 2.0) - jnp.log(idsf + 1.0)) * _INV_LOG_RANGE
    return jnp.log(q * float(NUM_SAMPLED))


def _select_rows(rows2, idc, hole_w):
    n = rows2.shape[0]
    q2 = idc >= 2 * _P
    rem = jnp.where(q2, idc - 2 * _P, idc)
    q1 = rem >= _P
    hl = jnp.where(q1, rows2[:, DIM:], rows2[:, :DIM])
    b = lax.bitcast_convert_type(hl, jnp.uint32)
    wb = jnp.where(q2, b & jnp.uint32(0xFFFF0000), b << 16)
    w = lax.bitcast_convert_type(wb, jnp.float32)
    oh = (lax.broadcasted_iota(jnp.int32, (n, _NHOLE), 1)
          == (idc - _TAIL)).astype(jnp.float32)
    hw = lax.dot_general(oh, hole_w, (((1,), (0,)), ((), ())),
                         preferred_element_type=jnp.float32)
    return jnp.where(idc >= _TAIL, hw, w)


def _tc_body(x_ref, tw2_ref, tb_ref, tid_ref, sw2_ref, sidc_ref, sb_ref,
             sid_ref, hole_ref, out_ref, sw_scr):
    @pl.when(pl.program_id(0) == 0)
    def _():
        w = _select_rows(sw2_ref[...], sidc_ref[...], hole_ref[...][:, :DIM])
        idf = sidc_ref[...].astype(jnp.float32)
        bc = sb_ref[...] - _log_expected_count(idf)
        b0 = bc.astype(jnp.bfloat16).astype(jnp.float32)
        r1 = bc - b0
        b1 = r1.astype(jnp.bfloat16).astype(jnp.float32)
        b2 = r1 - b1
        col = lax.broadcasted_iota(jnp.int32, (NUM_SAMPLED, 2 * DIM), 1)
        aug = jnp.where(col < DIM, jnp.pad(w, ((0, 0), (0, DIM))),
                        jnp.where(col == DIM, b0,
                                  jnp.where(col == DIM + 1, b1,
                                            jnp.where(col == DIM + 2, b2,
                                                      0.0))))
        sw_scr[...] = aug.astype(jnp.bfloat16)

    sid = sid_ref[...]
    x = x_ref[...]
    logits = lax.dot_general(
        x, sw_scr[...], (((1,), (1,)), ((), ())),
        preferred_element_type=jnp.float32)
    tid = tid_ref[...]
    logits = jnp.where(tid == sid, logits - 1e9, logits)

    tw = _select_rows(tw2_ref[...], tid, hole_ref[...][:, :DIM])
    tl = (jnp.sum(x[:, :DIM].astype(jnp.float32) * tw, axis=1, keepdims=True)
          + tb_ref[...])
    tl = tl - _log_expected_count(tid.astype(jnp.float32))
    m = jnp.maximum(jnp.max(logits, axis=1, keepdims=True), tl)
    se = jnp.sum(jnp.exp(logits - m), axis=1, keepdims=True) + jnp.exp(tl - m)
    out_ref[...] = jnp.log(se) + m - tl


def _tc_loss(x_aug, rows2, tb, tid, sidc, sb, sid, hole_w):
    ts = NUM_SAMPLED // _BLK
    return pl.pallas_call(
        _tc_body,
        grid=(BATCH // _BLK,),
        compiler_params=_sc_params(dimension_semantics=("arbitrary",)),
        in_specs=[
            pl.BlockSpec((_BLK, 2 * DIM), lambda i: (i, 0)),
            pl.BlockSpec((_BLK, 2 * DIM), lambda i: (i + ts, 0)),
            pl.BlockSpec((_BLK, 1), lambda i: (i, 0)),
            pl.BlockSpec((_BLK, 1), lambda i: (i, 0)),
            pl.BlockSpec((NUM_SAMPLED, 2 * DIM), lambda i: (0, 0)),
            pl.BlockSpec((NUM_SAMPLED, 1), lambda i: (0, 0)),
            pl.BlockSpec((NUM_SAMPLED, 1), lambda i: (0, 0)),
            pl.BlockSpec((1, NUM_SAMPLED), lambda i: (0, 0)),
            pl.BlockSpec((_NHOLE, 2 * DIM), lambda i: (0, 0)),
        ],
        out_specs=pl.BlockSpec((_BLK, 1), lambda i: (i, 0)),
        out_shape=jax.ShapeDtypeStruct((BATCH, 1), jnp.float32),
        scratch_shapes=[pltpu.VMEM((NUM_SAMPLED, 2 * DIM), jnp.bfloat16)],
    )(x_aug, rows2, tb, tid, rows2, sidc, sb, sid, hole_w)


def kernel(inputs, labels, kernel, bias, sampled_ids):
    table_t = kernel.T
    packed, hole_w = _tc_relayout(table_t)
    ids_all = jnp.concatenate([sampled_ids, labels[:, 0]])
    rows2 = _sc_rows(packed, ids_all)
    bvals = _sc_bias(bias.reshape(NUM_CLASSES // _L, _L), ids_all)
    sb = bvals[:NUM_SAMPLED].reshape(1, NUM_SAMPLED)
    tb = bvals[NUM_SAMPLED:].reshape(BATCH, 1)
    sid = sampled_ids.reshape(1, NUM_SAMPLED)
    sidc = sampled_ids.reshape(NUM_SAMPLED, 1)
    x_bf = inputs.astype(jnp.bfloat16)
    x_aug = jnp.concatenate(
        [x_bf, jnp.ones((BATCH, 3), jnp.bfloat16),
         jnp.zeros((BATCH, DIM - 3), jnp.bfloat16)], axis=1)
    out = _tc_loss(x_aug, rows2, tb, labels, sidc,
                   sb.reshape(NUM_SAMPLED, 1), sid, hole_w)
    return out[:, 0]
